# baseline (speedup 1.0000x reference)
import functools

import jax
import jax.numpy as jnp
from jax import lax
from jax.experimental import pallas as pl
from jax.experimental.pallas import tpu as pltpu
from jax.experimental.pallas import tpu_sc as plsc

H = 1024
DFF = 512
E = 8
K = 2
SCALE = 2.5
T = 2048
B = 512
NBR = 16
PR = NBR * B


BT = 256


def _router_body(x_ref, gw_ref, w_ref, i_ref, p_ref, c_ref, carry):
    b = pl.program_id(0)

    @pl.when(b == 0)
    def _():
        carry[...] = jnp.zeros_like(carry)

    x = x_ref[...]
    logits = lax.dot_general(x, gw_ref[...], (((1,), (1,)), ((), ())),
                             preferred_element_type=jnp.float32)
    v = jax.nn.sigmoid(logits)
    lane = lax.broadcasted_iota(jnp.int32, v.shape, 1)
    m1 = jnp.max(v, axis=1, keepdims=True)
    i1 = jnp.min(jnp.where(v == m1, lane, E), axis=1, keepdims=True)
    vm = jnp.where(lane == i1, -jnp.inf, v)
    m2 = jnp.max(vm, axis=1, keepdims=True)
    i2 = jnp.min(jnp.where(vm == m2, lane, E), axis=1, keepdims=True)
    s = m1 + m2 + 1e-6
    w_ref[...] = jnp.concatenate([m1 / s, m2 / s], axis=1) * SCALE
    i_ref[...] = jnp.concatenate([i1, i2], axis=1)

    a0 = (lane == i1).astype(jnp.float32)
    a1 = (lane == i2).astype(jnp.float32)
    sm = a0 + a1
    r = lax.broadcasted_iota(jnp.int32, (BT, BT), 0)
    c = lax.broadcasted_iota(jnp.int32, (BT, BT), 1)
    tri = (c < r).astype(jnp.float32)
    pfx = lax.dot_general(tri, sm, (((1,), (0,)), ((), ())),
                          preferred_element_type=jnp.float32) + carry[...]
    pos0 = jnp.sum(pfx * a0, axis=1, keepdims=True)
    pos1 = jnp.sum(pfx * a1, axis=1, keepdims=True)
    p_ref[...] = jnp.concatenate([pos0, pos1], axis=1).astype(jnp.int32)
    carry[...] = carry[...] + jnp.sum(sm, axis=0, keepdims=True)
    c_ref[...] = carry[...].astype(jnp.int32)


def _router(x, gate_w):
    return pl.pallas_call(
        _router_body,
        grid=(T // BT,),
        in_specs=[pl.BlockSpec((BT, H), lambda b: (b, 0)),
                  pl.BlockSpec((E, H), lambda b: (0, 0))],
        out_specs=(pl.BlockSpec((BT, K), lambda b: (b, 0)),
                   pl.BlockSpec((BT, K), lambda b: (b, 0)),
                   pl.BlockSpec((BT, K), lambda b: (b, 0)),
                   pl.BlockSpec((1, E), lambda b: (0, 0))),
        out_shape=(jax.ShapeDtypeStruct((T, K), jnp.float32),
                   jax.ShapeDtypeStruct((T, K), jnp.int32),
                   jax.ShapeDtypeStruct((T, K), jnp.int32),
                   jax.ShapeDtypeStruct((1, E), jnp.int32)),
        scratch_shapes=[pltpu.VMEM((1, E), jnp.float32)],
    )(x, gate_w)


def _make_sc_dispatch():
    info = plsc.get_sparse_core_info()
    nw = info.num_cores * info.num_subcores
    nt = T // nw
    mesh = plsc.VectorSubcoreMesh(core_axis_name="c", subcore_axis_name="s")

    @functools.partial(
        pl.kernel, mesh=mesh, name="sc_dispatch_scatter",
        out_type=jax.ShapeDtypeStruct((PR, H), jnp.float32),
        scratch_types=[
            pltpu.VMEM((nt, H), jnp.float32),
            pltpu.VMEM((nt,), jnp.int32),
            pltpu.VMEM((nt,), jnp.int32),
            pltpu.SemaphoreType.DMA,
        ],
    )
    def dispatch_kernel(x_hbm, d0_hbm, d1_hbm, out_hbm, xv, i0v, i1v, sem):
        wid = lax.axis_index("s") * info.num_cores + lax.axis_index("c")
        base = wid * nt
        pltpu.sync_copy(x_hbm.at[pl.ds(base, nt)], xv)
        pltpu.sync_copy(d0_hbm.at[pl.ds(base, nt)], i0v)
        pltpu.sync_copy(d1_hbm.at[pl.ds(base, nt)], i1v)
        c0 = pltpu.async_copy(xv, out_hbm.at[i0v], sem)
        c1 = pltpu.async_copy(xv, out_hbm.at[i1v], sem)
        c0.wait()
        c1.wait()

    return dispatch_kernel


def _make_sc_gather(n_rows, n_chunks, name, dtype=jnp.float32):
    info = plsc.get_sparse_core_info()
    nw = info.num_cores * info.num_subcores
    n_w = n_rows // nw
    chunk = n_w // n_chunks
    mesh = plsc.VectorSubcoreMesh(core_axis_name="c", subcore_axis_name="s")

    @functools.partial(
        pl.kernel, mesh=mesh, name=name,
        out_type=jax.ShapeDtypeStruct((n_rows, H), dtype),
        scratch_types=[
            pltpu.VMEM((n_w,), jnp.int32),
            pltpu.VMEM((chunk, H), dtype),
            pltpu.VMEM((chunk, H), dtype),
            pltpu.SemaphoreType.DMA,
            pltpu.SemaphoreType.DMA,
        ],
    )
    def gather_kernel(src_hbm, idx_hbm, out_hbm, idx_v, rows0, rows1, s0, s1):
        wid = lax.axis_index("s") * info.num_cores + lax.axis_index("c")
        base = wid * n_w
        bufs, sems = (rows0, rows1), (s0, s1)
        pltpu.sync_copy(idx_hbm.at[pl.ds(base, n_w)], idx_v)

        def start(c):
            return pltpu.async_copy(
                src_hbm.at[idx_v.at[pl.ds(c * chunk, chunk)]],
                bufs[c % 2], sems[c % 2])

        cps = [None] * n_chunks
        cps[0] = start(0)
        if n_chunks > 1:
            cps[1] = start(1)
        for c in range(n_chunks):
            cps[c].wait()
            pltpu.sync_copy(bufs[c % 2],
                            out_hbm.at[pl.ds(base + c * chunk, chunk)])
            if c + 2 < n_chunks:
                cps[c + 2] = start(c + 2)

    return gather_kernel


NBS = NBR + T // B


def _mlp_compute(xb, gw16, uw16, dw16, y_ref):
    g = lax.dot_general(xb, gw16[...], (((1,), (1,)), ((), ())),
                        preferred_element_type=jnp.float32)
    u = lax.dot_general(xb, uw16[...], (((1,), (1,)), ((), ())),
                        preferred_element_type=jnp.float32)
    h = (jax.nn.silu(g) * u).astype(jnp.bfloat16)
    y_ref[...] = lax.dot_general(h, dw16[...], (((1,), (1,)), ((), ())),
                                 preferred_element_type=jnp.float32)


def _gmm(meta, xg, x, gw, uw, dw, sgw, suw, sdw):
    grid_spec = pltpu.PrefetchScalarGridSpec(
        num_scalar_prefetch=1,
        grid=(NBS + 1,),
        in_specs=[
            pl.BlockSpec(
                (B, H),
                lambda s, m: (jnp.clip(s - 1, 0, NBR - 1), 0)),
            pl.BlockSpec(
                (B, H),
                lambda s, m: (jnp.maximum(s - 1 - NBR, 0), 0)),
            pl.BlockSpec((1, DFF, H), lambda s, m: (m[3, s], 0, 0)),
            pl.BlockSpec((1, DFF, H), lambda s, m: (m[3, s], 0, 0)),
            pl.BlockSpec((1, H, DFF), lambda s, m: (m[3, s], 0, 0)),
            pl.BlockSpec((1, DFF, H), lambda s, m: (0, 0, 0)),
            pl.BlockSpec((1, DFF, H), lambda s, m: (0, 0, 0)),
            pl.BlockSpec((1, H, DFF), lambda s, m: (0, 0, 0)),
        ],
        out_specs=pl.BlockSpec((B, H), lambda s, m: (jnp.maximum(s - 1, 0), 0)),
        scratch_shapes=[
            pltpu.VMEM((DFF, H), jnp.bfloat16),
            pltpu.VMEM((DFF, H), jnp.bfloat16),
            pltpu.VMEM((H, DFF), jnp.bfloat16),
            pltpu.VMEM((DFF, H), jnp.bfloat16),
            pltpu.VMEM((DFF, H), jnp.bfloat16),
            pltpu.VMEM((H, DFF), jnp.bfloat16),
            pltpu.VMEM((B, H), jnp.bfloat16),
        ],
    )

    def body(m_ref, xg_ref, x_ref, gw_ref, uw_ref, dw_ref,
             sgw_ref, suw_ref, sdw_ref, y_ref,
             g0, u0, d0, g1, u1, d1, xb16):
        s = pl.program_id(0)
        e = m_ref[0, s]
        ce = m_ref[1, s]
        us = m_ref[2, s]
        cs = m_ref[4, s]

        @pl.when((ce >= 0) & (ce < E) & (cs == 0))
        def _c0():
            g0[...] = gw_ref[0].astype(jnp.bfloat16)
            u0[...] = uw_ref[0].astype(jnp.bfloat16)
            d0[...] = dw_ref[0].astype(jnp.bfloat16)

        @pl.when((ce >= 0) & (ce < E) & (cs == 1))
        def _c1():
            g1[...] = gw_ref[0].astype(jnp.bfloat16)
            u1[...] = uw_ref[0].astype(jnp.bfloat16)
            d1[...] = dw_ref[0].astype(jnp.bfloat16)

        @pl.when((ce == E) & (cs == 0))
        def _cs0():
            g0[...] = sgw_ref[0].astype(jnp.bfloat16)
            u0[...] = suw_ref[0].astype(jnp.bfloat16)
            d0[...] = sdw_ref[0].astype(jnp.bfloat16)

        @pl.when((ce == E) & (cs == 1))
        def _cs1():
            g1[...] = sgw_ref[0].astype(jnp.bfloat16)
            u1[...] = suw_ref[0].astype(jnp.bfloat16)
            d1[...] = sdw_ref[0].astype(jnp.bfloat16)

        @pl.when((e >= 0) & (e < E))
        def _ld_routed():
            xb16[...] = xg_ref[...].astype(jnp.bfloat16)

        @pl.when(e == E)
        def _ld_shared():
            xb16[...] = x_ref[...].astype(jnp.bfloat16)

        @pl.when((e >= 0) & (us == 0))
        def _mlp0():
            _mlp_compute(xb16[...], g0, u0, d0, y_ref)

        @pl.when((e >= 0) & (us == 1))
        def _mlp1():
            _mlp_compute(xb16[...], g1, u1, d1, y_ref)

    return pl.pallas_call(
        body,
        grid_spec=grid_spec,
        out_shape=jax.ShapeDtypeStruct((PR + T, H), jnp.float32),
    )(meta, xg, x, gw, uw, dw, sgw, suw, sdw)


def _combine_body(g0_ref, g1_ref, sh_ref, w_ref, o_ref):
    w = w_ref[...]
    o_ref[...] = (w[:, 0:1] * g0_ref[...].astype(jnp.float32)
                  + w[:, 1:2] * g1_ref[...].astype(jnp.float32)
                  + sh_ref[...])


def _combine(g, yg, topk_w):
    bt = 256
    return pl.pallas_call(
        _combine_body,
        grid=(T // bt,),
        in_specs=[pl.BlockSpec((bt, H), lambda i: (i, 0)),
                  pl.BlockSpec((bt, H), lambda i: (i + T // bt, 0)),
                  pl.BlockSpec((bt, H), lambda i: (i + PR // bt, 0)),
                  pl.BlockSpec((bt, K), lambda i: (i, 0))],
        out_specs=pl.BlockSpec((bt, H), lambda i: (i, 0)),
        out_shape=jax.ShapeDtypeStruct((T, H), jnp.float32),
    )(g, g, yg, topk_w)


def kernel(hidden_states, gate_w, shared_gate_w, shared_up_w, shared_down_w,
           expert_gate_w, expert_up_w, expert_down_w):
    orig_shape = hidden_states.shape
    x = hidden_states.reshape(-1, H)

    topk_w, topk_i, pos, counts2d = _router(x, gate_w)

    flat_e = topk_i.reshape(-1)
    counts = counts2d[0]
    pad_counts = ((counts + B - 1) // B) * B
    pad_off = jnp.concatenate([jnp.zeros((1,), jnp.int32),
                               jnp.cumsum(pad_counts)]).astype(jnp.int32)
    dest = pad_off[flat_e] + pos.reshape(-1)

    b_start = jnp.arange(NBR, dtype=jnp.int32) * B
    block_expert = jnp.where(
        b_start < pad_off[E],
        jnp.minimum(
            jnp.sum((b_start[:, None] >= pad_off[None, 1:E + 1])
                    .astype(jnp.int32), axis=1), E - 1),
        -1)
    block_expert = jnp.concatenate(
        [block_expert, jnp.full((T // B,), E, jnp.int32)])

    d_cols = dest.reshape(T, K)
    xg = _make_sc_dispatch()(x, d_cols[:, 0], d_cols[:, 1])

    comp_e = jnp.concatenate([jnp.full((1,), -9, jnp.int32), block_expert])
    nxt = jnp.concatenate([comp_e[1:], jnp.full((1,), -1, jnp.int32)])
    cast_e = jnp.where((nxt != comp_e) & (nxt >= 0), nxt, -1)
    ev = jnp.cumsum((cast_e >= 0).astype(jnp.int32))
    cast_set = ev % 2
    use_set = jnp.concatenate([jnp.zeros((1,), jnp.int32), ev[:-1] % 2])
    windex = jnp.clip(
        lax.cummax(jnp.where(cast_e >= 0, cast_e, -1), axis=0), 0, E - 1)
    meta = jnp.stack([comp_e, cast_e, use_set, windex, cast_set])
    yg = _gmm(meta, xg, x,
              expert_gate_w, expert_up_w, expert_down_w,
              shared_gate_w[None], shared_up_w[None], shared_down_w[None])

    d_planar = dest.reshape(T, K).T.reshape(-1)
    g = _make_sc_gather(T * K, 8, "sc_gather_combine")(yg, d_planar)

    out = _combine(g, yg, topk_w)
    return out.reshape(orig_shape)


# speedup vs baseline: 1.0018x; 1.0018x over previous
import functools

import jax
import jax.numpy as jnp
from jax import lax
from jax.experimental import pallas as pl
from jax.experimental.pallas import tpu as pltpu
from jax.experimental.pallas import tpu_sc as plsc

H = 1024
DFF = 512
E = 8
K = 2
SCALE = 2.5
T = 2048
B = 512
NBR = 16
PR = NBR * B


BT = 256


def _router_body(x_ref, gw_ref, w_ref, i_ref, p_ref, c_ref, carry):
    b = pl.program_id(0)

    @pl.when(b == 0)
    def _():
        carry[...] = jnp.zeros_like(carry)

    x = x_ref[...]
    logits = lax.dot_general(x, gw_ref[...], (((1,), (1,)), ((), ())),
                             preferred_element_type=jnp.float32)
    v = jax.nn.sigmoid(logits)
    lane = lax.broadcasted_iota(jnp.int32, v.shape, 1)
    m1 = jnp.max(v, axis=1, keepdims=True)
    i1 = jnp.min(jnp.where(v == m1, lane, E), axis=1, keepdims=True)
    vm = jnp.where(lane == i1, -jnp.inf, v)
    m2 = jnp.max(vm, axis=1, keepdims=True)
    i2 = jnp.min(jnp.where(vm == m2, lane, E), axis=1, keepdims=True)
    s = m1 + m2 + 1e-6
    w_ref[...] = jnp.concatenate([m1 / s, m2 / s], axis=1) * SCALE
    i_ref[...] = jnp.concatenate([i1, i2], axis=1)

    a0 = (lane == i1).astype(jnp.float32)
    a1 = (lane == i2).astype(jnp.float32)
    sm = a0 + a1
    r = lax.broadcasted_iota(jnp.int32, (BT, BT), 0)
    c = lax.broadcasted_iota(jnp.int32, (BT, BT), 1)
    tri = (c < r).astype(jnp.float32)
    pfx = lax.dot_general(tri, sm, (((1,), (0,)), ((), ())),
                          preferred_element_type=jnp.float32) + carry[...]
    pos0 = jnp.sum(pfx * a0, axis=1, keepdims=True)
    pos1 = jnp.sum(pfx * a1, axis=1, keepdims=True)
    p_ref[...] = jnp.concatenate([pos0, pos1], axis=1).astype(jnp.int32)
    carry[...] = carry[...] + jnp.sum(sm, axis=0, keepdims=True)
    c_ref[...] = carry[...].astype(jnp.int32)


def _router(x, gate_w):
    return pl.pallas_call(
        _router_body,
        grid=(T // BT,),
        in_specs=[pl.BlockSpec((BT, H), lambda b: (b, 0)),
                  pl.BlockSpec((E, H), lambda b: (0, 0))],
        out_specs=(pl.BlockSpec((BT, K), lambda b: (b, 0)),
                   pl.BlockSpec((BT, K), lambda b: (b, 0)),
                   pl.BlockSpec((BT, K), lambda b: (b, 0)),
                   pl.BlockSpec((1, E), lambda b: (0, 0))),
        out_shape=(jax.ShapeDtypeStruct((T, K), jnp.float32),
                   jax.ShapeDtypeStruct((T, K), jnp.int32),
                   jax.ShapeDtypeStruct((T, K), jnp.int32),
                   jax.ShapeDtypeStruct((1, E), jnp.int32)),
        scratch_shapes=[pltpu.VMEM((1, E), jnp.float32)],
    )(x, gate_w)


def _make_sc_dispatch():
    info = plsc.get_sparse_core_info()
    nw = info.num_cores * info.num_subcores
    nt = T // nw
    mesh = plsc.VectorSubcoreMesh(core_axis_name="c", subcore_axis_name="s")

    @functools.partial(
        pl.kernel, mesh=mesh, name="sc_dispatch_scatter",
        out_type=jax.ShapeDtypeStruct((PR, H), jnp.float32),
        scratch_types=[
            pltpu.VMEM((nt, H), jnp.float32),
            pltpu.VMEM((nt,), jnp.int32),
            pltpu.VMEM((nt,), jnp.int32),
            pltpu.SemaphoreType.DMA,
        ],
    )
    def dispatch_kernel(x_hbm, d0_hbm, d1_hbm, out_hbm, xv, i0v, i1v, sem):
        wid = lax.axis_index("s") * info.num_cores + lax.axis_index("c")
        base = wid * nt
        pltpu.sync_copy(x_hbm.at[pl.ds(base, nt)], xv)
        pltpu.sync_copy(d0_hbm.at[pl.ds(base, nt)], i0v)
        pltpu.sync_copy(d1_hbm.at[pl.ds(base, nt)], i1v)
        c0 = pltpu.async_copy(xv, out_hbm.at[i0v], sem)
        c1 = pltpu.async_copy(xv, out_hbm.at[i1v], sem)
        c0.wait()
        c1.wait()

    return dispatch_kernel


def _make_sc_gather(n_rows, n_chunks, name, dtype=jnp.float32):
    info = plsc.get_sparse_core_info()
    nw = info.num_cores * info.num_subcores
    n_w = n_rows // nw
    chunk = n_w // n_chunks
    mesh = plsc.VectorSubcoreMesh(core_axis_name="c", subcore_axis_name="s")

    @functools.partial(
        pl.kernel, mesh=mesh, name=name,
        out_type=jax.ShapeDtypeStruct((n_rows, H), dtype),
        scratch_types=[
            pltpu.VMEM((n_w,), jnp.int32),
            pltpu.VMEM((chunk, H), dtype),
            pltpu.VMEM((chunk, H), dtype),
            pltpu.SemaphoreType.DMA,
            pltpu.SemaphoreType.DMA,
        ],
    )
    def gather_kernel(src_hbm, idx_hbm, out_hbm, idx_v, rows0, rows1, s0, s1):
        wid = lax.axis_index("s") * info.num_cores + lax.axis_index("c")
        base = wid * n_w
        bufs, sems = (rows0, rows1), (s0, s1)
        pltpu.sync_copy(idx_hbm.at[pl.ds(base, n_w)], idx_v)

        def start(c):
            return pltpu.async_copy(
                src_hbm.at[idx_v.at[pl.ds(c * chunk, chunk)]],
                bufs[c % 2], sems[c % 2])

        cps = [None] * n_chunks
        cps[0] = start(0)
        if n_chunks > 1:
            cps[1] = start(1)
        for c in range(n_chunks):
            cps[c].wait()
            pltpu.sync_copy(bufs[c % 2],
                            out_hbm.at[pl.ds(base + c * chunk, chunk)])
            if c + 2 < n_chunks:
                cps[c + 2] = start(c + 2)

    return gather_kernel


NBS = NBR + T // B


def _mlp_compute(xb, gw16, uw16, dw16, y_ref):
    g = lax.dot_general(xb, gw16[...], (((1,), (1,)), ((), ())),
                        preferred_element_type=jnp.float32)
    u = lax.dot_general(xb, uw16[...], (((1,), (1,)), ((), ())),
                        preferred_element_type=jnp.float32)
    h = (jax.nn.silu(g) * u).astype(jnp.bfloat16)
    y_ref[...] = lax.dot_general(h, dw16[...], (((1,), (1,)), ((), ())),
                                 preferred_element_type=jnp.float32)


def _gmm(meta, xg, x, gw, uw, dw, sgw, suw, sdw):
    grid_spec = pltpu.PrefetchScalarGridSpec(
        num_scalar_prefetch=1,
        grid=(NBS + 1,),
        in_specs=[
            pl.BlockSpec(
                (B, H),
                lambda s, m: (jnp.clip(s - 1, 0, NBR - 1), 0)),
            pl.BlockSpec(
                (B, H),
                lambda s, m: (jnp.maximum(s - 1 - NBR, 0), 0)),
            pl.BlockSpec((1, DFF, H), lambda s, m: (m[3, s], 0, 0)),
            pl.BlockSpec((1, DFF, H), lambda s, m: (m[3, s], 0, 0)),
            pl.BlockSpec((1, H, DFF), lambda s, m: (m[3, s], 0, 0)),
            pl.BlockSpec((1, DFF, H), lambda s, m: (0, 0, 0)),
            pl.BlockSpec((1, DFF, H), lambda s, m: (0, 0, 0)),
            pl.BlockSpec((1, H, DFF), lambda s, m: (0, 0, 0)),
        ],
        out_specs=pl.BlockSpec((B, H), lambda s, m: (jnp.maximum(s - 1, 0), 0)),
        scratch_shapes=[
            pltpu.VMEM((DFF, H), jnp.bfloat16),
            pltpu.VMEM((DFF, H), jnp.bfloat16),
            pltpu.VMEM((H, DFF), jnp.bfloat16),
            pltpu.VMEM((DFF, H), jnp.bfloat16),
            pltpu.VMEM((DFF, H), jnp.bfloat16),
            pltpu.VMEM((H, DFF), jnp.bfloat16),
            pltpu.VMEM((B, H), jnp.bfloat16),
        ],
    )

    def body(m_ref, xg_ref, x_ref, gw_ref, uw_ref, dw_ref,
             sgw_ref, suw_ref, sdw_ref, y_ref,
             g0, u0, d0, g1, u1, d1, xb16):
        s = pl.program_id(0)
        e = m_ref[0, s]
        ce = m_ref[1, s]
        us = m_ref[2, s]
        cs = m_ref[4, s]

        @pl.when((ce >= 0) & (ce < E) & (cs == 0))
        def _c0():
            g0[...] = gw_ref[0].astype(jnp.bfloat16)
            u0[...] = uw_ref[0].astype(jnp.bfloat16)
            d0[...] = dw_ref[0].astype(jnp.bfloat16)

        @pl.when((ce >= 0) & (ce < E) & (cs == 1))
        def _c1():
            g1[...] = gw_ref[0].astype(jnp.bfloat16)
            u1[...] = uw_ref[0].astype(jnp.bfloat16)
            d1[...] = dw_ref[0].astype(jnp.bfloat16)

        @pl.when((ce == E) & (cs == 0))
        def _cs0():
            g0[...] = sgw_ref[0].astype(jnp.bfloat16)
            u0[...] = suw_ref[0].astype(jnp.bfloat16)
            d0[...] = sdw_ref[0].astype(jnp.bfloat16)

        @pl.when((ce == E) & (cs == 1))
        def _cs1():
            g1[...] = sgw_ref[0].astype(jnp.bfloat16)
            u1[...] = suw_ref[0].astype(jnp.bfloat16)
            d1[...] = sdw_ref[0].astype(jnp.bfloat16)

        @pl.when((e >= 0) & (e < E))
        def _ld_routed():
            xb16[...] = xg_ref[...].astype(jnp.bfloat16)

        @pl.when(e == E)
        def _ld_shared():
            xb16[...] = x_ref[...].astype(jnp.bfloat16)

        @pl.when((e >= 0) & (us == 0))
        def _mlp0():
            _mlp_compute(xb16[...], g0, u0, d0, y_ref)

        @pl.when((e >= 0) & (us == 1))
        def _mlp1():
            _mlp_compute(xb16[...], g1, u1, d1, y_ref)

    return pl.pallas_call(
        body,
        grid_spec=grid_spec,
        out_shape=jax.ShapeDtypeStruct((PR + T, H), jnp.float32),
    )(meta, xg, x, gw, uw, dw, sgw, suw, sdw)


def _combine_body(g0_ref, g1_ref, sh_ref, w_ref, o_ref):
    w = w_ref[...]
    o_ref[...] = (w[:, 0:1] * g0_ref[...].astype(jnp.float32)
                  + w[:, 1:2] * g1_ref[...].astype(jnp.float32)
                  + sh_ref[...])


def _combine(g, yg, topk_w):
    bt = 256
    return pl.pallas_call(
        _combine_body,
        grid=(T // bt,),
        in_specs=[pl.BlockSpec((bt, H), lambda i: (i, 0)),
                  pl.BlockSpec((bt, H), lambda i: (i + T // bt, 0)),
                  pl.BlockSpec((bt, H), lambda i: (i + PR // bt, 0)),
                  pl.BlockSpec((bt, K), lambda i: (i, 0))],
        out_specs=pl.BlockSpec((bt, H), lambda i: (i, 0)),
        out_shape=jax.ShapeDtypeStruct((T, H), jnp.float32),
    )(g, g, yg, topk_w)


def kernel(hidden_states, gate_w, shared_gate_w, shared_up_w, shared_down_w,
           expert_gate_w, expert_up_w, expert_down_w):
    orig_shape = hidden_states.shape
    x = hidden_states.reshape(-1, H)

    topk_w, topk_i, pos, counts2d = _router(x, gate_w)

    flat_e = topk_i.reshape(-1)
    counts = counts2d[0]
    pad_counts = ((counts + B - 1) // B) * B
    pad_off = jnp.concatenate([jnp.zeros((1,), jnp.int32),
                               jnp.cumsum(pad_counts)]).astype(jnp.int32)
    dest = pad_off[flat_e] + pos.reshape(-1)

    b_start = jnp.arange(NBR, dtype=jnp.int32) * B
    block_expert = jnp.where(
        b_start < pad_off[E],
        jnp.minimum(
            jnp.sum((b_start[:, None] >= pad_off[None, 1:E + 1])
                    .astype(jnp.int32), axis=1), E - 1),
        -1)
    block_expert = jnp.concatenate(
        [block_expert, jnp.full((T // B,), E, jnp.int32)])

    d_cols = dest.reshape(T, K)
    xg = _make_sc_dispatch()(x, d_cols[:, 0], d_cols[:, 1])

    comp_e = jnp.concatenate([jnp.full((1,), -9, jnp.int32), block_expert])
    nxt = jnp.concatenate([comp_e[1:], jnp.full((1,), -1, jnp.int32)])
    cast_e = jnp.where((nxt != comp_e) & (nxt >= 0), nxt, -1)
    ev = jnp.cumsum((cast_e >= 0).astype(jnp.int32))
    cast_set = ev % 2
    use_set = jnp.concatenate([jnp.zeros((1,), jnp.int32), ev[:-1] % 2])
    windex = jnp.clip(
        lax.cummax(jnp.where(cast_e >= 0, cast_e, -1), axis=0), 0, E - 1)
    meta = jnp.stack([comp_e, cast_e, use_set, windex, cast_set])
    yg = _gmm(meta, xg, x,
              expert_gate_w, expert_up_w, expert_down_w,
              shared_gate_w[None], shared_up_w[None], shared_down_w[None])

    d_planar = dest.reshape(T, K).T.reshape(-1)
    g = _make_sc_gather(T * K, 4, "sc_gather_combine")(yg, d_planar)

    out = _combine(g, yg, topk_w)
    return out.reshape(orig_shape)
